# baseline (speedup 1.0000x reference)
import jax
import jax.numpy as jnp
from jax.experimental import pallas as pl

NUM_NODE = 8
NUM_ITER = 4
D = 128
C = 32
B = 4096
BLK = 512
EC = NUM_NODE * C


def _fused_kernel(label_ref, img_ref, We_ref, be_ref, Wdt_ref, sel_ref,
                  bd_ref, loss_ref, imgs_ref, codes_ref):
    img = img_ref[...]
    lab = label_ref[...]
    We = We_ref[...]
    be = be_ref[...]
    Wdt = Wdt_ref[...]
    bd = bd_ref[...]

    lane_expert = jax.lax.broadcasted_iota(jnp.int32, (BLK, EC), 1) // C
    maskf = (lane_expert == lab).astype(jnp.float32)

    x = img * 2.0 - 1.0
    recon = jnp.zeros_like(img)
    lsum = jnp.float32(0.0)
    for i in range(NUM_ITER):
        enc = jnp.dot(x, We, preferred_element_type=jnp.float32) + be
        hardm = jnp.where(enc > 0, maskf, 0.0)
        dec = jnp.tanh(
            jnp.dot(hardm, Wdt, preferred_element_type=jnp.float32) + bd)
        if i == 0:
            dec = (dec + 1.0) * 0.5
        recon = recon + dec
        diff = recon - img
        lsum = lsum + jnp.sum(diff * diff)
        x = -diff
        imgs_ref[i] = recon
        codes_ref[:, i * C:(i + 1) * C] = jnp.dot(
            hardm, sel_ref[0, :, :C], preferred_element_type=jnp.float32)

    b = pl.program_id(0)
    lsum2d = jnp.reshape(lsum, (1, 1))

    @pl.when(b == 0)
    def _init():
        loss_ref[...] = lsum2d

    @pl.when(b != 0)
    def _acc():
        loss_ref[...] += lsum2d


@jax.jit
def kernel(img, label, We, be, Wd, bd):
    label2d = label.astype(jnp.int32).reshape(B, 1)
    We_flat = We.transpose(1, 0, 2).reshape(D, EC)
    be_flat = be.reshape(1, EC)
    Wd_tile = jnp.tile(Wd, (NUM_NODE, 1))
    eye_t = jnp.tile(jnp.eye(C, dtype=jnp.float32), (NUM_NODE, 1))
    sel = jnp.stack([
        jnp.pad(eye_t, ((0, 0), (i * C, (NUM_ITER - 1 - i) * C)))
        for i in range(NUM_ITER)
    ])
    bd2d = bd.reshape(1, D)

    grid = (B // BLK,)
    loss_sum, imgs, codes = pl.pallas_call(
        _fused_kernel,
        grid=grid,
        in_specs=[
            pl.BlockSpec((BLK, 1), lambda b: (b, 0)),
            pl.BlockSpec((BLK, D), lambda b: (b, 0)),
            pl.BlockSpec((D, EC), lambda b: (0, 0)),
            pl.BlockSpec((1, EC), lambda b: (0, 0)),
            pl.BlockSpec((EC, D), lambda b: (0, 0)),
            pl.BlockSpec((NUM_ITER, EC, NUM_ITER * C), lambda b: (0, 0, 0)),
            pl.BlockSpec((1, D), lambda b: (0, 0)),
        ],
        out_specs=[
            pl.BlockSpec((1, 1), lambda b: (0, 0)),
            pl.BlockSpec((NUM_ITER, BLK, D), lambda b: (0, b, 0)),
            pl.BlockSpec((BLK, NUM_ITER * C), lambda b: (b, 0)),
        ],
        out_shape=[
            jax.ShapeDtypeStruct((1, 1), jnp.float32),
            jax.ShapeDtypeStruct((NUM_ITER, B, D), jnp.float32),
            jax.ShapeDtypeStruct((B, NUM_ITER * C), jnp.float32),
        ],
    )(label2d, img, We_flat, be_flat, Wd_tile, sel, bd2d)

    loss = loss_sum[0, 0] / jnp.float32(B * D * NUM_ITER)
    return loss, imgs, codes


# speedup vs baseline: 1.0824x; 1.0824x over previous
import jax
import jax.numpy as jnp
from jax.experimental import pallas as pl

NUM_NODE = 8
NUM_ITER = 4
D = 128
C = 32
B = 4096
BLK = 2048
EC = NUM_NODE * C


def _fused_kernel(label_ref, img_ref, We_ref, be_ref, Wdt_ref, sel_ref,
                  bd_ref, loss_ref, imgs_ref, codes_ref):
    img = img_ref[...]
    lab = label_ref[...]
    We = We_ref[...]
    be = be_ref[...]
    Wdt = Wdt_ref[...]
    bd = bd_ref[...]

    lane_expert = jax.lax.broadcasted_iota(jnp.int32, (BLK, EC), 1) // C
    maskf = (lane_expert == lab).astype(jnp.float32)

    x = img * 2.0 - 1.0
    recon = jnp.zeros_like(img)
    lsum = jnp.float32(0.0)
    for i in range(NUM_ITER):
        enc = jnp.dot(x, We, preferred_element_type=jnp.float32) + be
        hardm = jnp.where(enc > 0, maskf, 0.0)
        dec = jnp.tanh(
            jnp.dot(hardm, Wdt, preferred_element_type=jnp.float32) + bd)
        if i == 0:
            dec = (dec + 1.0) * 0.5
        recon = recon + dec
        diff = recon - img
        lsum = lsum + jnp.sum(diff * diff)
        x = -diff
        imgs_ref[i] = recon
        codes_ref[:, i * C:(i + 1) * C] = jnp.dot(
            hardm, sel_ref[0, :, :C], preferred_element_type=jnp.float32)

    b = pl.program_id(0)
    lsum2d = jnp.reshape(lsum, (1, 1))

    @pl.when(b == 0)
    def _init():
        loss_ref[...] = lsum2d

    @pl.when(b != 0)
    def _acc():
        loss_ref[...] += lsum2d


@jax.jit
def kernel(img, label, We, be, Wd, bd):
    label2d = label.astype(jnp.int32).reshape(B, 1)
    We_flat = We.transpose(1, 0, 2).reshape(D, EC)
    be_flat = be.reshape(1, EC)
    Wd_tile = jnp.tile(Wd, (NUM_NODE, 1))
    eye_t = jnp.tile(jnp.eye(C, dtype=jnp.float32), (NUM_NODE, 1))
    sel = jnp.stack([
        jnp.pad(eye_t, ((0, 0), (i * C, (NUM_ITER - 1 - i) * C)))
        for i in range(NUM_ITER)
    ])
    bd2d = bd.reshape(1, D)

    grid = (B // BLK,)
    loss_sum, imgs, codes = pl.pallas_call(
        _fused_kernel,
        grid=grid,
        in_specs=[
            pl.BlockSpec((BLK, 1), lambda b: (b, 0)),
            pl.BlockSpec((BLK, D), lambda b: (b, 0)),
            pl.BlockSpec((D, EC), lambda b: (0, 0)),
            pl.BlockSpec((1, EC), lambda b: (0, 0)),
            pl.BlockSpec((EC, D), lambda b: (0, 0)),
            pl.BlockSpec((NUM_ITER, EC, NUM_ITER * C), lambda b: (0, 0, 0)),
            pl.BlockSpec((1, D), lambda b: (0, 0)),
        ],
        out_specs=[
            pl.BlockSpec((1, 1), lambda b: (0, 0)),
            pl.BlockSpec((NUM_ITER, BLK, D), lambda b: (0, b, 0)),
            pl.BlockSpec((BLK, NUM_ITER * C), lambda b: (b, 0)),
        ],
        out_shape=[
            jax.ShapeDtypeStruct((1, 1), jnp.float32),
            jax.ShapeDtypeStruct((NUM_ITER, B, D), jnp.float32),
            jax.ShapeDtypeStruct((B, NUM_ITER * C), jnp.float32),
        ],
    )(label2d, img, We_flat, be_flat, Wd_tile, sel, bd2d)

    loss = loss_sum[0, 0] / jnp.float32(B * D * NUM_ITER)
    return loss, imgs, codes


# speedup vs baseline: 1.3589x; 1.2554x over previous
import numpy as np

import jax
import jax.numpy as jnp
from jax.experimental import pallas as pl

NUM_NODE = 8
NUM_ITER = 4
D = 128
C = 32
B = 4096
BLK = 2048
EC = NUM_NODE * C

_FOLD = np.tile(np.eye(C, dtype=np.float32), (NUM_NODE, 1))
_LOSS_SCALE = np.float32(1.0 / (B * D * NUM_ITER))


def _fused_kernel(label_ref, img_ref, We_ref, be_ref, Wd_ref,
                  bd_ref, fold_ref, loss_ref, imgs_ref, codes_ref):
    img = img_ref[...]
    lab = label_ref[...]
    We = We_ref[...]
    be = be_ref[...]
    Wd = Wd_ref[...]
    bd = bd_ref[...]
    fold = fold_ref[...]

    lane_expert = jax.lax.broadcasted_iota(jnp.int32, (BLK, EC), 1) // C
    maskf = (lane_expert == lab).astype(jnp.float32)

    x = img * 2.0 - 1.0
    recon = jnp.zeros_like(img)
    lsum = jnp.float32(0.0)
    for i in range(NUM_ITER):
        enc = jnp.dot(x, We, preferred_element_type=jnp.float32) + be
        hardm = jnp.where(enc > 0, maskf, 0.0)
        hard = jnp.dot(hardm, fold, preferred_element_type=jnp.float32)
        dec = jnp.tanh(
            jnp.dot(hard, Wd, preferred_element_type=jnp.float32) + bd)
        if i == 0:
            dec = (dec + 1.0) * 0.5
        recon = recon + dec
        diff = recon - img
        lsum = lsum + jnp.sum(diff * diff)
        x = -diff
        imgs_ref[i] = recon
        codes_ref[:, i * C:(i + 1) * C] = hard

    b = pl.program_id(0)
    lsum2d = jnp.reshape(lsum * _LOSS_SCALE, (1, 1))

    @pl.when(b == 0)
    def _init():
        loss_ref[...] = lsum2d

    @pl.when(b != 0)
    def _acc():
        loss_ref[...] += lsum2d


@jax.jit
def kernel(img, label, We, be, Wd, bd):
    label2d = label.astype(jnp.int32).reshape(B, 1)
    We_flat = We.transpose(1, 0, 2).reshape(D, EC)
    be_flat = be.reshape(1, EC)
    bd2d = bd.reshape(1, D)

    grid = (B // BLK,)
    loss_sum, imgs, codes = pl.pallas_call(
        _fused_kernel,
        grid=grid,
        in_specs=[
            pl.BlockSpec((BLK, 1), lambda b: (b, 0)),
            pl.BlockSpec((BLK, D), lambda b: (b, 0)),
            pl.BlockSpec((D, EC), lambda b: (0, 0)),
            pl.BlockSpec((1, EC), lambda b: (0, 0)),
            pl.BlockSpec((C, D), lambda b: (0, 0)),
            pl.BlockSpec((1, D), lambda b: (0, 0)),
            pl.BlockSpec((EC, C), lambda b: (0, 0)),
        ],
        out_specs=[
            pl.BlockSpec((1, 1), lambda b: (0, 0)),
            pl.BlockSpec((NUM_ITER, BLK, D), lambda b: (0, b, 0)),
            pl.BlockSpec((BLK, NUM_ITER * C), lambda b: (b, 0)),
        ],
        out_shape=[
            jax.ShapeDtypeStruct((1, 1), jnp.float32),
            jax.ShapeDtypeStruct((NUM_ITER, B, D), jnp.float32),
            jax.ShapeDtypeStruct((B, NUM_ITER * C), jnp.float32),
        ],
    )(label2d, img, We_flat, be_flat, Wd, bd2d, jnp.asarray(_FOLD))

    return loss_sum[0, 0], imgs, codes
